# baseline (speedup 1.0000x reference)
import jax
import jax.numpy as jnp
from jax.experimental import pallas as pl


def _copy_body(x_ref, o_ref):
    o_ref[...] = x_ref[...]


def kernel(x, emb):
    B, T, S, D = x.shape
    R = B * T * S
    RB = 1568
    x2 = x.reshape(R, D)
    out = pl.pallas_call(
        _copy_body,
        grid=(R // RB,),
        in_specs=[
            pl.BlockSpec((RB, D), lambda i: (i, 0)),
        ],
        out_specs=pl.BlockSpec((RB, D), lambda i: (i, 0)),
        out_shape=jax.ShapeDtypeStruct((R, D), x.dtype),
    )(x2)
    return out.reshape(B, T, S, D)


# speedup vs baseline: 1.8910x; 1.8910x over previous
import jax
import jax.numpy as jnp
from jax import lax
from jax.experimental import pallas as pl
from jax.experimental.pallas import tpu as pltpu

_NBUF = 8


def _body(x_hbm, emb_ref, o_hbm, inbuf, outbuf, in_sem, out_sem):
    B, T, S, D = x_hbm.shape
    N = B * T

    for c in range(_NBUF):
        pltpu.make_async_copy(
            x_hbm.at[c // T, c % T], inbuf.at[c], in_sem.at[c]
        ).start()

    def step(c, carry):
        slot = lax.rem(c, _NBUF)
        b = lax.div(c, T)
        t = lax.rem(c, T)
        pltpu.make_async_copy(
            x_hbm.at[b, t], inbuf.at[slot], in_sem.at[slot]
        ).wait()

        @pl.when(c >= _NBUF)
        def _():
            pc = c - _NBUF
            pltpu.make_async_copy(
                outbuf.at[slot],
                o_hbm.at[lax.div(pc, T), lax.rem(pc, T)],
                out_sem.at[slot],
            ).wait()

        outbuf[slot] = inbuf[slot] + emb_ref[t]
        pltpu.make_async_copy(
            outbuf.at[slot], o_hbm.at[b, t], out_sem.at[slot]
        ).start()

        nc = c + _NBUF

        @pl.when(nc < N)
        def _():
            pltpu.make_async_copy(
                x_hbm.at[lax.div(nc, T), lax.rem(nc, T)],
                inbuf.at[slot],
                in_sem.at[slot],
            ).start()

        return carry

    lax.fori_loop(0, N, step, 0)

    for k in range(_NBUF):
        c = N - _NBUF + k
        pltpu.make_async_copy(
            outbuf.at[c % _NBUF], o_hbm.at[c // T, c % T], out_sem.at[c % _NBUF]
        ).wait()


def kernel(x, emb):
    B, T, S, D = x.shape
    emb3 = emb.reshape(T, 1, D)
    return pl.pallas_call(
        _body,
        in_specs=[
            pl.BlockSpec(memory_space=pl.ANY),
            pl.BlockSpec(memory_space=pltpu.VMEM),
        ],
        out_specs=pl.BlockSpec(memory_space=pl.ANY),
        out_shape=jax.ShapeDtypeStruct(x.shape, x.dtype),
        scratch_shapes=[
            pltpu.VMEM((_NBUF, S, D), x.dtype),
            pltpu.VMEM((_NBUF, S, D), x.dtype),
            pltpu.SemaphoreType.DMA((_NBUF,)),
            pltpu.SemaphoreType.DMA((_NBUF,)),
        ],
    )(x, emb3)
